# Initial kernel scaffold; baseline (speedup 1.0000x reference)
import jax
import jax.numpy as jnp
from jax.experimental import pallas as pl


def kernel(input_ids, table):
    raise NotImplementedError("write your pallas kernel here")


# speedup vs baseline: 1.5441x; 1.5441x over previous
import functools

import jax
import jax.numpy as jnp
from jax import lax
from jax.experimental import pallas as pl
from jax.experimental.pallas import tpu as pltpu
from jax.experimental.pallas import tpu_sc as plsc

_NUM_CORES = 2
_NUM_SUBCORES = 16
_NUM_WORKERS = _NUM_CORES * _NUM_SUBCORES


@functools.lru_cache(maxsize=None)
def _make_gather(B, V, D):
    assert B % _NUM_WORKERS == 0
    b_per_w = B // _NUM_WORKERS
    C = 64
    assert b_per_w % C == 0
    n_chunks = b_per_w // C

    mesh = plsc.VectorSubcoreMesh(core_axis_name="c", subcore_axis_name="s")

    @functools.partial(
        pl.kernel,
        mesh=mesh,
        out_type=jax.ShapeDtypeStruct((B, D), jnp.float32),
        scratch_types=[
            pltpu.VMEM((b_per_w,), jnp.int32),
            pltpu.VMEM((C, D), jnp.float32),
            pltpu.SemaphoreType.DMA,
        ],
    )
    def gather_kernel(idx_hbm, table_hbm, out_hbm, idx_v, buf, sem):
        wid = lax.axis_index("s") * _NUM_CORES + lax.axis_index("c")
        base = wid * b_per_w
        pltpu.sync_copy(idx_hbm.at[pl.ds(base, b_per_w)], idx_v)
        for j in range(n_chunks):
            pltpu.async_copy(
                table_hbm.at[idx_v.at[pl.ds(j * C, C)]], buf, sem
            ).wait()
            pltpu.sync_copy(buf, out_hbm.at[pl.ds(base + j * C, C)])

    return gather_kernel


def kernel(input_ids, table):
    ids = jnp.reshape(input_ids, (-1,)).astype(jnp.int32)
    B = ids.shape[0]
    V, D = table.shape
    out = _make_gather(B, V, D)(ids, table)
    return out.reshape(input_ids.shape[1], input_ids.shape[2], D)


# speedup vs baseline: 1.6462x; 1.0661x over previous
import functools

import jax
import jax.numpy as jnp
from jax import lax
from jax.experimental import pallas as pl
from jax.experimental.pallas import tpu as pltpu
from jax.experimental.pallas import tpu_sc as plsc

_NUM_CORES = 2
_NUM_SUBCORES = 16
_NUM_WORKERS = _NUM_CORES * _NUM_SUBCORES


@functools.lru_cache(maxsize=None)
def _make_gather(B, V, D):
    assert B % _NUM_WORKERS == 0
    b_per_w = B // _NUM_WORKERS
    C = 32
    assert b_per_w % C == 0
    n_chunks = b_per_w // C

    mesh = plsc.VectorSubcoreMesh(core_axis_name="c", subcore_axis_name="s")

    @functools.partial(
        pl.kernel,
        mesh=mesh,
        out_type=jax.ShapeDtypeStruct((B, D), jnp.float32),
        scratch_types=[
            pltpu.VMEM((b_per_w,), jnp.int32),
            pltpu.VMEM((C, D), jnp.float32),
            pltpu.VMEM((C, D), jnp.float32),
            pltpu.SemaphoreType.DMA,
            pltpu.SemaphoreType.DMA,
            pltpu.SemaphoreType.DMA,
            pltpu.SemaphoreType.DMA,
        ],
    )
    def gather_kernel(idx_hbm, table_hbm, out_hbm, idx_v, buf0, buf1,
                      g0, g1, w0, w1):
        wid = lax.axis_index("s") * _NUM_CORES + lax.axis_index("c")
        base = wid * b_per_w
        pltpu.sync_copy(idx_hbm.at[pl.ds(base, b_per_w)], idx_v)
        bufs = (buf0, buf1)
        gsems = (g0, g1)
        wsems = (w0, w1)
        gcopy = [None, None]
        wcopy = [None, None]
        gcopy[0] = pltpu.async_copy(
            table_hbm.at[idx_v.at[pl.ds(0, C)]], bufs[0], gsems[0])
        for j in range(n_chunks):
            b = j & 1
            nb = 1 - b
            if j + 1 < n_chunks:
                if wcopy[nb] is not None:
                    wcopy[nb].wait()
                gcopy[nb] = pltpu.async_copy(
                    table_hbm.at[idx_v.at[pl.ds((j + 1) * C, C)]],
                    bufs[nb], gsems[nb])
            gcopy[b].wait()
            wcopy[b] = pltpu.async_copy(
                bufs[b], out_hbm.at[pl.ds(base + j * C, C)], wsems[b])
        wcopy[0].wait()
        wcopy[1].wait()

    return gather_kernel


def kernel(input_ids, table):
    ids = jnp.reshape(input_ids, (-1,)).astype(jnp.int32)
    B = ids.shape[0]
    V, D = table.shape
    out = _make_gather(B, V, D)(ids, table)
    return out.reshape(input_ids.shape[1], input_ids.shape[2], D)


# speedup vs baseline: 1.6670x; 1.0127x over previous
import functools

import jax
import jax.numpy as jnp
from jax import lax
from jax.experimental import pallas as pl
from jax.experimental.pallas import tpu as pltpu
from jax.experimental.pallas import tpu_sc as plsc

_NUM_CORES = 2
_NUM_SUBCORES = 16
_NUM_WORKERS = _NUM_CORES * _NUM_SUBCORES


@functools.lru_cache(maxsize=None)
def _make_gather(B, V, D):
    assert B % _NUM_WORKERS == 0
    b_per_w = B // _NUM_WORKERS
    C = 32
    NBUF = 3
    assert b_per_w % C == 0
    n_chunks = b_per_w // C

    mesh = plsc.VectorSubcoreMesh(core_axis_name="c", subcore_axis_name="s")

    @functools.partial(
        pl.kernel,
        mesh=mesh,
        out_type=jax.ShapeDtypeStruct((B, D), jnp.float32),
        scratch_types=[
            pltpu.VMEM((b_per_w,), jnp.int32),
        ] + [pltpu.VMEM((C, D), jnp.float32)] * NBUF
          + [pltpu.SemaphoreType.DMA] * (2 * NBUF),
    )
    def gather_kernel(idx_hbm, table_hbm, out_hbm, idx_v, *rest):
        bufs = rest[:NBUF]
        gsems = rest[NBUF:2 * NBUF]
        wsems = rest[2 * NBUF:]
        wid = lax.axis_index("s") * _NUM_CORES + lax.axis_index("c")
        base = wid * b_per_w
        pltpu.sync_copy(idx_hbm.at[pl.ds(base, b_per_w)], idx_v)
        gcopy = [None] * NBUF
        wcopy = [None] * NBUF
        for j in range(min(NBUF, n_chunks)):
            gcopy[j] = pltpu.async_copy(
                table_hbm.at[idx_v.at[pl.ds(j * C, C)]], bufs[j], gsems[j])
        for j in range(n_chunks):
            b = j % NBUF
            gcopy[b].wait()
            wcopy[b] = pltpu.async_copy(
                bufs[b], out_hbm.at[pl.ds(base + j * C, C)], wsems[b])
            nj = j + NBUF
            if nj < n_chunks:
                wcopy[b].wait()
                gcopy[b] = pltpu.async_copy(
                    table_hbm.at[idx_v.at[pl.ds(nj * C, C)]],
                    bufs[b], gsems[b])
                wcopy[b] = None
        for b in range(NBUF):
            if wcopy[b] is not None:
                wcopy[b].wait()

    return gather_kernel


def kernel(input_ids, table):
    ids = jnp.reshape(input_ids, (-1,)).astype(jnp.int32)
    B = ids.shape[0]
    V, D = table.shape
    out = _make_gather(B, V, D)(ids, table)
    return out.reshape(input_ids.shape[1], input_ids.shape[2], D)
